# baseline (speedup 1.0000x reference)
import functools

import jax
import jax.numpy as jnp
from jax import lax
from jax.experimental import pallas as pl
from jax.experimental.pallas import tpu as pltpu
from jax.experimental.pallas import tpu_sc as plsc

N_NODES = 10000
N_EDGES = 320000
D_FEAT = 128
N_GRAPHS = 16
D_H = 1024
EPS = 1e-5

ROW_BLK = 1000

SC_TILES = 16
EROWS = 160
E_PAD = SC_TILES * EROWS * 128
NP = N_NODES + 16
CROWS = 10240


def _bdot(a, b):
    return jax.lax.dot_general(a.astype(jnp.bfloat16), b.astype(jnp.bfloat16),
                               (((1,), (0,)), ((), ())),
                               preferred_element_type=jnp.float32)


def _y_body(x_ref, w1_ref, b1_ref, y_ref, cs_ref, cs_acc):
    i = pl.program_id(0)
    nsteps = pl.num_programs(0)
    y = _bdot(x_ref[...], w1_ref[...]) + b1_ref[0:1, :]
    y_ref[...] = y

    @pl.when(i == 0)
    def _():
        cs_acc[...] = jnp.zeros_like(cs_acc)

    cs_acc[...] += jnp.sum(y, axis=0, keepdims=True)

    @pl.when(i == nsteps - 1)
    def _():
        cs_ref[...] = jnp.broadcast_to(cs_acc[...], cs_ref.shape)


def _y_pass(x, W1, b1r):
    return pl.pallas_call(
        _y_body,
        grid=(N_NODES // ROW_BLK,),
        in_specs=[
            pl.BlockSpec((ROW_BLK, D_FEAT), lambda i: (i, 0)),
            pl.BlockSpec((D_FEAT, D_H), lambda i: (0, 0)),
            pl.BlockSpec((1, D_H), lambda i: (0, 0)),
        ],
        out_specs=[
            pl.BlockSpec((ROW_BLK, D_H), lambda i: (i, 0)),
            pl.BlockSpec((8, D_H), lambda i: (0, 0)),
        ],
        out_shape=[
            jax.ShapeDtypeStruct((N_NODES, D_H), jnp.float32),
            jax.ShapeDtypeStruct((8, D_H), jnp.float32),
        ],
        scratch_shapes=[pltpu.VMEM((1, D_H), jnp.float32)],
    )(x, W1, b1r)


def _var_body(y_ref, mean_ref, var_ref, acc):
    i = pl.program_id(0)
    nsteps = pl.num_programs(0)

    @pl.when(i == 0)
    def _():
        acc[...] = jnp.zeros_like(acc)

    d = y_ref[...] - mean_ref[0:1, :]
    acc[...] += jnp.sum(d * d, axis=0, keepdims=True)

    @pl.when(i == nsteps - 1)
    def _():
        var_ref[...] = jnp.broadcast_to(acc[...] / jnp.float32(N_NODES),
                                        var_ref.shape)


def _var_pass(y, mean):
    return pl.pallas_call(
        _var_body,
        grid=(N_NODES // ROW_BLK,),
        in_specs=[
            pl.BlockSpec((ROW_BLK, D_H), lambda i: (i, 0)),
            pl.BlockSpec((1, D_H), lambda i: (0, 0)),
        ],
        out_specs=pl.BlockSpec((8, D_H), lambda i: (0, 0)),
        out_shape=jax.ShapeDtypeStruct((8, D_H), jnp.float32),
        scratch_shapes=[pltpu.VMEM((1, D_H), jnp.float32)],
    )(y, mean)


def _hp_body(y_ref, mean_ref, var_ref, g_ref, be_ref, wp_ref, h_ref, p_ref):
    h = jnp.maximum(
        g_ref[0:1, :] * (y_ref[...] - mean_ref[0:1, :])
        / jnp.sqrt(var_ref[0:1, :] + EPS) + be_ref[0:1, :], 0.0)
    h_ref[...] = h
    p_ref[...] = _bdot(h, wp_ref[...])


def _h_and_p(y, mean, var, g1r, be1r, Wp2):
    return pl.pallas_call(
        _hp_body,
        grid=(N_NODES // ROW_BLK,),
        in_specs=[
            pl.BlockSpec((ROW_BLK, D_H), lambda i: (i, 0)),
            pl.BlockSpec((1, D_H), lambda i: (0, 0)),
            pl.BlockSpec((1, D_H), lambda i: (0, 0)),
            pl.BlockSpec((1, D_H), lambda i: (0, 0)),
            pl.BlockSpec((1, D_H), lambda i: (0, 0)),
            pl.BlockSpec((D_H, 128), lambda i: (0, 0)),
        ],
        out_specs=[
            pl.BlockSpec((ROW_BLK, D_H), lambda i: (i, 0)),
            pl.BlockSpec((ROW_BLK, 128), lambda i: (i, 0)),
        ],
        out_shape=[
            jax.ShapeDtypeStruct((N_NODES, D_H), jnp.float32),
            jax.ShapeDtypeStruct((N_NODES, 128), jnp.float32),
        ],
    )(y, mean, var, g1r, be1r, Wp2)


def _segment_softmax(e, seg, num_segments):
    m = jax.ops.segment_max(e, seg, num_segments=num_segments)
    m = jnp.where(jnp.isfinite(m), m, 0.0)
    ex = jnp.exp(e - m[seg])
    denom = jax.ops.segment_sum(ex, seg, num_segments=num_segments)
    return ex / (denom[seg] + 1e-16)


def _sc_edge_body(src_hbm, dst_hbm, p0_hbm, p1_hbm, out_hbm,
                  srcv, dstv, exv, outv, p0v, p1v, denv, den_sh):
    w = lax.axis_index("s")
    base = w * EROWS
    pltpu.sync_copy(src_hbm.at[pl.ds(base, EROWS)], srcv)
    pltpu.sync_copy(dst_hbm.at[pl.ds(base, EROWS)], dstv)
    pltpu.sync_copy(p0_hbm, p0v)
    pltpu.sync_copy(p1_hbm, p1v)

    def zbody(i, carry):
        denv[pl.ds(i * 16, 16)] = jnp.zeros((16,), jnp.float32)
        return carry
    lax.fori_loop(0, NP // 16, zbody, 0)

    @pl.when(w == 0)
    def _():
        pltpu.sync_copy(denv, den_sh)
    plsc.subcore_barrier()

    def gbody(r, carry):
        for k in range(8):
            sv = srcv[r, pl.ds(k * 16, 16)]
            dv = dstv[r, pl.ds(k * 16, 16)]
            pa = plsc.load_gather(p0v, [sv])
            pb = plsc.load_gather(p1v, [dv])
            exv[r, pl.ds(k * 16, 16)] = jnp.exp(pa + pb)
        return carry
    lax.fori_loop(0, EROWS, gbody, 0)

    def sbody(r, carry):
        pltpu.sync_copy(exv.at[r], den_sh.at[dstv.at[r]], add=True)
        return carry
    lax.fori_loop(0, EROWS, sbody, 0)
    plsc.subcore_barrier()

    pltpu.sync_copy(den_sh, denv)

    def nbody(r, carry):
        for k in range(8):
            dv = dstv[r, pl.ds(k * 16, 16)]
            den = plsc.load_gather(denv, [dv])
            ev = exv[r, pl.ds(k * 16, 16)]
            outv[r, pl.ds(k * 16, 16)] = ev / (den + 1e-16) + 0.5
        return carry
    lax.fori_loop(0, EROWS, nbody, 0)
    pltpu.sync_copy(outv, out_hbm.at[pl.ds(base, EROWS)])


def _sc_edge(srcm, dstm, p0p, p1p):
    mesh = plsc.VectorSubcoreMesh(core_axis_name="c", subcore_axis_name="s",
                                  num_cores=1)
    f = functools.partial(
        pl.kernel,
        mesh=mesh,
        compiler_params=pltpu.CompilerParams(needs_layout_passes=False),
        out_type=jax.ShapeDtypeStruct((SC_TILES * EROWS, 128), jnp.float32),
        scratch_types=[
            pltpu.VMEM((EROWS, 128), jnp.int32),
            pltpu.VMEM((EROWS, 128), jnp.int32),
            pltpu.VMEM((EROWS, 128), jnp.float32),
            pltpu.VMEM((EROWS, 128), jnp.float32),
            pltpu.VMEM((NP,), jnp.float32),
            pltpu.VMEM((NP,), jnp.float32),
            pltpu.VMEM((NP,), jnp.float32),
            pltpu.VMEM_SHARED((NP,), jnp.float32),
        ],
    )(_sc_edge_body)
    return f(srcm, dstm, p0p, p1p)


def _sc_newx_body(h_hbm, ia_hbm, ib_hbm, out_hbm,
                  iav, ibv, rowsA, rowsB, semA, semB):
    w = lax.axis_index("s")

    def chunk(c, carry):
        rowbase = w * (CROWS // SC_TILES) + c * 32
        pltpu.sync_copy(ia_hbm.at[pl.ds(rowbase, 32)], iav)
        pltpu.sync_copy(ib_hbm.at[pl.ds(rowbase, 32)], ibv)
        cpA = pltpu.async_copy(h_hbm.at[iav], rowsA, semA)
        cpB = pltpu.async_copy(h_hbm.at[ibv], rowsB, semB)
        cpA.wait()
        cpB.wait()

        def radd(r, carry2):
            for k in range(64):
                a = rowsA[r, pl.ds(k * 16, 16)]
                b = rowsB[r, pl.ds(k * 16, 16)]
                rowsA[r, pl.ds(k * 16, 16)] = a + b
            return carry2
        lax.fori_loop(0, 32, radd, 0)
        pltpu.sync_copy(rowsA, out_hbm.at[pl.ds(rowbase, 32)])
        return carry
    lax.fori_loop(0, (CROWS // SC_TILES) // 32, chunk, 0)


def _sc_newx(h, iap, ibp):
    mesh = plsc.VectorSubcoreMesh(core_axis_name="c", subcore_axis_name="s",
                                  num_cores=1)
    f = functools.partial(
        pl.kernel,
        mesh=mesh,
        out_type=jax.ShapeDtypeStruct((CROWS, D_H), jnp.float32),
        scratch_types=[
            pltpu.VMEM((32,), jnp.int32),
            pltpu.VMEM((32,), jnp.int32),
            pltpu.VMEM((32, D_H), jnp.float32),
            pltpu.VMEM((32, D_H), jnp.float32),
            pltpu.SemaphoreType.DMA,
            pltpu.SemaphoreType.DMA,
        ],
    )(_sc_newx_body)
    return f(h, iap, ibp)


def _merge_rounds(e, src, dst):
    ebits = jax.lax.bitcast_convert_type(e, jnp.int32)
    idx = jnp.arange(N_EDGES, dtype=jnp.int32)

    def cond(state):
        remaining, partner, escore, r = state
        return jnp.any(remaining[src] & remaining[dst])

    def body(state):
        remaining, partner, escore, r = state
        alive = remaining[src] & remaining[dst]
        ab = jnp.where(alive, ebits, -1)
        best = jnp.full((N_NODES,), -1, jnp.int32).at[src].max(ab).at[dst].max(ab)
        cs = alive & (ebits == best[src])
        cd = alive & (ebits == best[dst])
        bidx = jnp.full((N_NODES,), N_EDGES, jnp.int32)
        bidx = bidx.at[src].min(jnp.where(cs, idx, N_EDGES))
        bidx = bidx.at[dst].min(jnp.where(cd, idx, N_EDGES))
        take = (bidx[src] == idx) & (bidx[dst] == idx)
        ts = jnp.where(take, src, N_NODES)
        td = jnp.where(take, dst, N_NODES)
        partner = partner.at[ts].set(dst, mode="drop")
        partner = partner.at[td].set(src, mode="drop")
        escore = escore.at[ts].set(e, mode="drop")
        escore = escore.at[td].set(e, mode="drop")
        remaining = remaining.at[ts].set(False, mode="drop")
        remaining = remaining.at[td].set(False, mode="drop")
        return remaining, partner, escore, r + 1

    remaining0 = jnp.ones((N_NODES,), bool)
    partner0 = jnp.arange(N_NODES, dtype=jnp.int32)
    escore0 = jnp.ones((N_NODES,), e.dtype)
    remaining, partner, escore, _ = jax.lax.while_loop(
        cond, body, (remaining0, partner0, escore0, jnp.int32(0)))
    return partner, escore


def _clusters(partner, escore):
    v = jnp.arange(N_NODES, dtype=jnp.int32)
    rep = v <= partner
    cid = jnp.cumsum(rep.astype(jnp.int32)) - 1
    C = jnp.sum(rep.astype(jnp.int32))
    nodeA = jnp.zeros((N_NODES,), jnp.int32).at[
        jnp.where(rep, cid, N_NODES)].set(v, mode="drop")
    nodeB = partner[nodeA]
    sc = escore[nodeA]
    rowscale = jnp.where(jnp.arange(N_NODES) < C,
                         jnp.where(nodeB == nodeA, sc * 0.5, sc), 0.0)
    return nodeA, nodeB, rowscale, C


TBLK = 1024


def _pre_body(nx_ref, rs_ref, w2_ref, b2_ref, c_ref, pre_ref, cs_ref, acc):
    i = pl.program_id(0)
    nsteps = pl.num_programs(0)
    rs = rs_ref[...][:, 0:1]
    pre = _bdot(nx_ref[...] * rs, w2_ref[...]) + b2_ref[0:1, :]
    pre_ref[...] = pre

    @pl.when(i == 0)
    def _():
        acc[...] = jnp.zeros_like(acc)

    rid = i * TBLK + jax.lax.broadcasted_iota(jnp.int32, (TBLK, 1), 0)
    m = rid < c_ref[0, 0]
    acc[...] += jnp.sum(jnp.where(m, pre, 0.0), axis=0, keepdims=True)

    @pl.when(i == nsteps - 1)
    def _():
        cs_ref[...] = jnp.broadcast_to(acc[...], cs_ref.shape)


def _pre_pass(nx, rs128, W2, b2r, c11):
    return pl.pallas_call(
        _pre_body,
        grid=(CROWS // TBLK,),
        in_specs=[
            pl.BlockSpec((TBLK, D_H), lambda i: (i, 0)),
            pl.BlockSpec((TBLK, 128), lambda i: (i, 0)),
            pl.BlockSpec((D_H, 512), lambda i: (0, 0)),
            pl.BlockSpec((1, 512), lambda i: (0, 0)),
            pl.BlockSpec(memory_space=pltpu.SMEM),
        ],
        out_specs=[
            pl.BlockSpec((TBLK, 512), lambda i: (i, 0)),
            pl.BlockSpec((8, 512), lambda i: (0, 0)),
        ],
        out_shape=[
            jax.ShapeDtypeStruct((CROWS, 512), jnp.float32),
            jax.ShapeDtypeStruct((8, 512), jnp.float32),
        ],
        scratch_shapes=[pltpu.VMEM((1, 512), jnp.float32)],
    )(nx, rs128, W2, b2r, c11)


def _var2_body(pre_ref, mean_ref, c_ref, var_ref, acc):
    i = pl.program_id(0)
    nsteps = pl.num_programs(0)

    @pl.when(i == 0)
    def _():
        acc[...] = jnp.zeros_like(acc)

    rid = i * TBLK + jax.lax.broadcasted_iota(jnp.int32, (TBLK, 1), 0)
    m = rid < c_ref[0, 0]
    d = pre_ref[...] - mean_ref[0:1, :]
    acc[...] += jnp.sum(jnp.where(m, d * d, 0.0), axis=0, keepdims=True)

    @pl.when(i == nsteps - 1)
    def _():
        var_ref[...] = jnp.broadcast_to(
            acc[...] / c_ref[0, 0].astype(jnp.float32), var_ref.shape)


def _var2_pass(pre, mean, c11):
    return pl.pallas_call(
        _var2_body,
        grid=(CROWS // TBLK,),
        in_specs=[
            pl.BlockSpec((TBLK, 512), lambda i: (i, 0)),
            pl.BlockSpec((1, 512), lambda i: (0, 0)),
            pl.BlockSpec(memory_space=pltpu.SMEM),
        ],
        out_specs=pl.BlockSpec((8, 512), lambda i: (0, 0)),
        out_shape=jax.ShapeDtypeStruct((8, 512), jnp.float32),
        scratch_shapes=[pltpu.VMEM((1, 512), jnp.float32)],
    )(pre, mean, c11)


def _pool_body(pre_ref, mean_ref, var_ref, g2_ref, be2_ref, oh_ref,
               wfc_ref, bfc_ref, wfc1_ref, bfc1_ref, z_ref, pacc, cacc):
    i = pl.program_id(0)
    nsteps = pl.num_programs(0)

    @pl.when(i == 0)
    def _():
        pacc[...] = jnp.zeros_like(pacc)
        cacc[...] = jnp.zeros_like(cacc)

    h2 = jnp.maximum(
        g2_ref[0:1, :] * (pre_ref[...] - mean_ref[0:1, :])
        / jnp.sqrt(var_ref[0:1, :] + EPS) + be2_ref[0:1, :], 0.0)
    oh = oh_ref[...]
    pacc[...] += jax.lax.dot_general(
        oh, h2, (((0,), (0,)), ((), ())),
        preferred_element_type=jnp.float32,
        precision=jax.lax.Precision.HIGHEST)
    cacc[...] += jax.lax.dot_general(
        oh, jnp.ones((TBLK, 1), jnp.float32), (((0,), (0,)), ((), ())),
        preferred_element_type=jnp.float32,
        precision=jax.lax.Precision.HIGHEST)

    @pl.when(i == nsteps - 1)
    def _():
        pooled = pacc[...] / jnp.maximum(cacc[...], 1.0)
        z1 = jnp.maximum(_bdot(pooled, wfc_ref[...]) + bfc_ref[0:1, :], 0.0)
        z2 = _bdot(z1, wfc1_ref[...]) + bfc1_ref[0:1, :]
        z_ref[...] = z2[0:N_GRAPHS, :]


def _pool_pass(pre, mean, var, g2r, be2r, oh, Wfc, bfcr, Wfc1, bfc1r):
    return pl.pallas_call(
        _pool_body,
        grid=(CROWS // TBLK,),
        in_specs=[
            pl.BlockSpec((TBLK, 512), lambda i: (i, 0)),
            pl.BlockSpec((1, 512), lambda i: (0, 0)),
            pl.BlockSpec((1, 512), lambda i: (0, 0)),
            pl.BlockSpec((1, 512), lambda i: (0, 0)),
            pl.BlockSpec((1, 512), lambda i: (0, 0)),
            pl.BlockSpec((TBLK, 128), lambda i: (i, 0)),
            pl.BlockSpec((512, 200), lambda i: (0, 0)),
            pl.BlockSpec((1, 200), lambda i: (0, 0)),
            pl.BlockSpec((200, 10), lambda i: (0, 0)),
            pl.BlockSpec((1, 10), lambda i: (0, 0)),
        ],
        out_specs=pl.BlockSpec((N_GRAPHS, 10), lambda i: (0, 0)),
        out_shape=jax.ShapeDtypeStruct((N_GRAPHS, 10), jnp.float32),
        scratch_shapes=[
            pltpu.VMEM((128, 512), jnp.float32),
            pltpu.VMEM((128, 1), jnp.float32),
        ],
    )(pre, mean, var, g2r, be2r, oh, Wfc, bfcr, Wfc1, bfc1r)


def kernel(x, edge_index, batch, W1, b1, g1, be1, Wp, bp, W2, b2, g2, be2,
           Wfc, bfc, Wfc1, bfc1):
    src = edge_index[0]
    dst = edge_index[1]

    b1r = b1.reshape(1, D_H)
    g1r = g1.reshape(1, D_H)
    be1r = be1.reshape(1, D_H)
    y, cs8 = _y_pass(x, W1, b1r)
    mean = (cs8[0:1] / jnp.float32(N_NODES))
    var8 = _var_pass(y, mean)

    Wp2 = jnp.zeros((D_H, 128), jnp.float32)
    Wp2 = Wp2.at[:, 0].set(Wp[:D_H, 0]).at[:, 1].set(Wp[D_H:, 0])
    h, ppad = _h_and_p(y, mean, var8[0:1], g1r, be1r, Wp2)
    p0 = ppad[:, 0]
    p1 = ppad[:, 1]

    padi = jnp.full((E_PAD - N_EDGES,), N_NODES, jnp.int32)
    srcm = jnp.concatenate([src, padi]).reshape(SC_TILES * EROWS, 128)
    dstm = jnp.concatenate([dst, padi]).reshape(SC_TILES * EROWS, 128)
    padp = jnp.full((NP - N_NODES,), -1000.0, jnp.float32)
    p0p = jnp.concatenate([p0 + bp[0], padp])
    p1p = jnp.concatenate([p1, padp])
    e = _sc_edge(srcm, dstm, p0p, p1p).reshape(E_PAD)[:N_EDGES]

    partner, escore = _merge_rounds(e, src, dst)
    nodeA, nodeB, rowscale, C = _clusters(partner, escore)

    padc = jnp.zeros((CROWS - N_NODES,), jnp.int32)
    iap = jnp.concatenate([nodeA, padc])
    ibp = jnp.concatenate([nodeB, padc])
    new_x_raw = _sc_newx(h, iap, ibp)

    rsp = jnp.concatenate([rowscale, padc.astype(jnp.float32)])
    rs128 = jnp.broadcast_to(rsp[:, None], (CROWS, 128))
    valid = jnp.arange(N_NODES) < C
    new_batch = jnp.where(valid, jnp.maximum(batch[nodeA], batch[nodeB]),
                          N_GRAPHS)
    nbp = jnp.concatenate([new_batch,
                           jnp.full((CROWS - N_NODES,), N_GRAPHS, jnp.int32)])
    oh = jnp.zeros((CROWS, 128), jnp.float32).at[:, :N_GRAPHS].set(
        (nbp[:, None] == jnp.arange(N_GRAPHS)[None, :]).astype(jnp.float32))

    c11 = C.reshape(1, 1)
    pre, cs8 = _pre_pass(new_x_raw, rs128, W2, b2.reshape(1, 512), c11)
    mean2 = cs8[0:1] / C.astype(jnp.float32)
    var2 = _var2_pass(pre, mean2, c11)
    return _pool_pass(pre, mean2, var2[0:1], g2.reshape(1, 512),
                      be2.reshape(1, 512), oh, Wfc, bfc.reshape(1, 200),
                      Wfc1, bfc1.reshape(1, 10))


# speedup vs baseline: 78.4758x; 78.4758x over previous
import functools

import jax
import jax.numpy as jnp
from jax import lax
from jax.experimental import pallas as pl
from jax.experimental.pallas import tpu as pltpu
from jax.experimental.pallas import tpu_sc as plsc

N_NODES = 10000
N_EDGES = 320000
D_FEAT = 128
N_GRAPHS = 16
D_H = 1024
EPS = 1e-5

ROW_BLK = 1000

SC_TILES = 16
EROWS = 160
E_PAD = SC_TILES * EROWS * 128
NP = N_NODES + 16
CROWS = 10240


def _bdot(a, b):
    return jax.lax.dot_general(a.astype(jnp.bfloat16), b.astype(jnp.bfloat16),
                               (((1,), (0,)), ((), ())),
                               preferred_element_type=jnp.float32)


def _y_body(x_ref, w1_ref, b1_ref, y_ref, cs_ref, cs_acc):
    i = pl.program_id(0)
    nsteps = pl.num_programs(0)
    y = _bdot(x_ref[...], w1_ref[...]) + b1_ref[0:1, :]
    y_ref[...] = y

    @pl.when(i == 0)
    def _():
        cs_acc[...] = jnp.zeros_like(cs_acc)

    cs_acc[...] += jnp.sum(y, axis=0, keepdims=True)

    @pl.when(i == nsteps - 1)
    def _():
        cs_ref[...] = jnp.broadcast_to(cs_acc[...], cs_ref.shape)


def _y_pass(x, W1, b1r):
    return pl.pallas_call(
        _y_body,
        grid=(N_NODES // ROW_BLK,),
        in_specs=[
            pl.BlockSpec((ROW_BLK, D_FEAT), lambda i: (i, 0)),
            pl.BlockSpec((D_FEAT, D_H), lambda i: (0, 0)),
            pl.BlockSpec((1, D_H), lambda i: (0, 0)),
        ],
        out_specs=[
            pl.BlockSpec((ROW_BLK, D_H), lambda i: (i, 0)),
            pl.BlockSpec((8, D_H), lambda i: (0, 0)),
        ],
        out_shape=[
            jax.ShapeDtypeStruct((N_NODES, D_H), jnp.float32),
            jax.ShapeDtypeStruct((8, D_H), jnp.float32),
        ],
        scratch_shapes=[pltpu.VMEM((1, D_H), jnp.float32)],
    )(x, W1, b1r)


def _var_body(y_ref, mean_ref, var_ref, acc):
    i = pl.program_id(0)
    nsteps = pl.num_programs(0)

    @pl.when(i == 0)
    def _():
        acc[...] = jnp.zeros_like(acc)

    d = y_ref[...] - mean_ref[0:1, :]
    acc[...] += jnp.sum(d * d, axis=0, keepdims=True)

    @pl.when(i == nsteps - 1)
    def _():
        var_ref[...] = jnp.broadcast_to(acc[...] / jnp.float32(N_NODES),
                                        var_ref.shape)


def _var_pass(y, mean):
    return pl.pallas_call(
        _var_body,
        grid=(N_NODES // ROW_BLK,),
        in_specs=[
            pl.BlockSpec((ROW_BLK, D_H), lambda i: (i, 0)),
            pl.BlockSpec((1, D_H), lambda i: (0, 0)),
        ],
        out_specs=pl.BlockSpec((8, D_H), lambda i: (0, 0)),
        out_shape=jax.ShapeDtypeStruct((8, D_H), jnp.float32),
        scratch_shapes=[pltpu.VMEM((1, D_H), jnp.float32)],
    )(y, mean)


def _hp_body(y_ref, mean_ref, var_ref, g_ref, be_ref, wp_ref, h_ref, p_ref):
    h = jnp.maximum(
        g_ref[0:1, :] * (y_ref[...] - mean_ref[0:1, :])
        / jnp.sqrt(var_ref[0:1, :] + EPS) + be_ref[0:1, :], 0.0)
    h_ref[...] = h
    p_ref[...] = _bdot(h, wp_ref[...])


def _h_and_p(y, mean, var, g1r, be1r, Wp2):
    return pl.pallas_call(
        _hp_body,
        grid=(N_NODES // ROW_BLK,),
        in_specs=[
            pl.BlockSpec((ROW_BLK, D_H), lambda i: (i, 0)),
            pl.BlockSpec((1, D_H), lambda i: (0, 0)),
            pl.BlockSpec((1, D_H), lambda i: (0, 0)),
            pl.BlockSpec((1, D_H), lambda i: (0, 0)),
            pl.BlockSpec((1, D_H), lambda i: (0, 0)),
            pl.BlockSpec((D_H, 128), lambda i: (0, 0)),
        ],
        out_specs=[
            pl.BlockSpec((ROW_BLK, D_H), lambda i: (i, 0)),
            pl.BlockSpec((ROW_BLK, 128), lambda i: (i, 0)),
        ],
        out_shape=[
            jax.ShapeDtypeStruct((N_NODES, D_H), jnp.float32),
            jax.ShapeDtypeStruct((N_NODES, 128), jnp.float32),
        ],
    )(y, mean, var, g1r, be1r, Wp2)


def _segment_softmax(e, seg, num_segments):
    m = jax.ops.segment_max(e, seg, num_segments=num_segments)
    m = jnp.where(jnp.isfinite(m), m, 0.0)
    ex = jnp.exp(e - m[seg])
    denom = jax.ops.segment_sum(ex, seg, num_segments=num_segments)
    return ex / (denom[seg] + 1e-16)


def _sc_edge_body(src_hbm, dst_hbm, p0_hbm, p1_hbm, out_hbm,
                  srcv, dstv, exv, outv, p0v, p1v, denv, den_sh):
    w = lax.axis_index("s")
    base = w * EROWS
    pltpu.sync_copy(src_hbm.at[pl.ds(base, EROWS)], srcv)
    pltpu.sync_copy(dst_hbm.at[pl.ds(base, EROWS)], dstv)
    pltpu.sync_copy(p0_hbm, p0v)
    pltpu.sync_copy(p1_hbm, p1v)

    def zbody(i, carry):
        denv[pl.ds(i * 16, 16)] = jnp.zeros((16,), jnp.float32)
        return carry
    lax.fori_loop(0, NP // 16, zbody, 0)

    @pl.when(w == 0)
    def _():
        pltpu.sync_copy(denv, den_sh)
    plsc.subcore_barrier()

    def gbody(r, carry):
        for k in range(8):
            sv = srcv[r, pl.ds(k * 16, 16)]
            dv = dstv[r, pl.ds(k * 16, 16)]
            pa = plsc.load_gather(p0v, [sv])
            pb = plsc.load_gather(p1v, [dv])
            exv[r, pl.ds(k * 16, 16)] = jnp.exp(pa + pb)
        return carry
    lax.fori_loop(0, EROWS, gbody, 0)

    def sbody(r, carry):
        pltpu.sync_copy(exv.at[r], den_sh.at[dstv.at[r]], add=True)
        return carry
    lax.fori_loop(0, EROWS, sbody, 0)
    plsc.subcore_barrier()

    pltpu.sync_copy(den_sh, denv)

    def nbody(r, carry):
        for k in range(8):
            dv = dstv[r, pl.ds(k * 16, 16)]
            den = plsc.load_gather(denv, [dv])
            ev = exv[r, pl.ds(k * 16, 16)]
            outv[r, pl.ds(k * 16, 16)] = ev / (den + 1e-16) + 0.5
        return carry
    lax.fori_loop(0, EROWS, nbody, 0)
    pltpu.sync_copy(outv, out_hbm.at[pl.ds(base, EROWS)])


def _sc_edge(srcm, dstm, p0p, p1p):
    mesh = plsc.VectorSubcoreMesh(core_axis_name="c", subcore_axis_name="s",
                                  num_cores=1)
    f = functools.partial(
        pl.kernel,
        mesh=mesh,
        compiler_params=pltpu.CompilerParams(needs_layout_passes=False),
        out_type=jax.ShapeDtypeStruct((SC_TILES * EROWS, 128), jnp.float32),
        scratch_types=[
            pltpu.VMEM((EROWS, 128), jnp.int32),
            pltpu.VMEM((EROWS, 128), jnp.int32),
            pltpu.VMEM((EROWS, 128), jnp.float32),
            pltpu.VMEM((EROWS, 128), jnp.float32),
            pltpu.VMEM((NP,), jnp.float32),
            pltpu.VMEM((NP,), jnp.float32),
            pltpu.VMEM((NP,), jnp.float32),
            pltpu.VMEM_SHARED((NP,), jnp.float32),
        ],
    )(_sc_edge_body)
    return f(srcm, dstm, p0p, p1p)


def _sc_newx_body(h_hbm, ia_hbm, ib_hbm, out_hbm,
                  iav, ibv, rowsA, rowsB, semA, semB):
    w = lax.axis_index("s")

    def chunk(c, carry):
        rowbase = w * (CROWS // SC_TILES) + c * 32
        pltpu.sync_copy(ia_hbm.at[pl.ds(rowbase, 32)], iav)
        pltpu.sync_copy(ib_hbm.at[pl.ds(rowbase, 32)], ibv)
        cpA = pltpu.async_copy(h_hbm.at[iav], rowsA, semA)
        cpB = pltpu.async_copy(h_hbm.at[ibv], rowsB, semB)
        cpA.wait()
        cpB.wait()

        def radd(r, carry2):
            for k in range(64):
                a = rowsA[r, pl.ds(k * 16, 16)]
                b = rowsB[r, pl.ds(k * 16, 16)]
                rowsA[r, pl.ds(k * 16, 16)] = a + b
            return carry2
        lax.fori_loop(0, 32, radd, 0)
        pltpu.sync_copy(rowsA, out_hbm.at[pl.ds(rowbase, 32)])
        return carry
    lax.fori_loop(0, (CROWS // SC_TILES) // 32, chunk, 0)


def _sc_newx(h, iap, ibp):
    mesh = plsc.VectorSubcoreMesh(core_axis_name="c", subcore_axis_name="s",
                                  num_cores=1)
    f = functools.partial(
        pl.kernel,
        mesh=mesh,
        out_type=jax.ShapeDtypeStruct((CROWS, D_H), jnp.float32),
        scratch_types=[
            pltpu.VMEM((32,), jnp.int32),
            pltpu.VMEM((32,), jnp.int32),
            pltpu.VMEM((32, D_H), jnp.float32),
            pltpu.VMEM((32, D_H), jnp.float32),
            pltpu.SemaphoreType.DMA,
            pltpu.SemaphoreType.DMA,
        ],
    )(_sc_newx_body)
    return f(h, iap, ibp)


def _merge_rounds(e, src, dst):
    order = jnp.argsort(-e, stable=True)
    pri = jnp.zeros((N_EDGES,), jnp.int32).at[order].set(
        jnp.arange(N_EDGES, dtype=jnp.int32))

    idx = jnp.arange(N_EDGES, dtype=jnp.int32)
    nodes2 = jnp.concatenate([src, dst])
    pri2 = jnp.concatenate([pri, pri])
    other2 = jnp.concatenate([dst, src])
    eid2 = jnp.concatenate([idx, idx])
    nodes_s, _, other_s, eid_s = jax.lax.sort(
        (nodes2, pri2, other2, eid2), num_keys=2)
    vs = jnp.arange(N_NODES, dtype=jnp.int32)
    start = jnp.searchsorted(nodes_s, vs).astype(jnp.int32)
    end = jnp.searchsorted(nodes_s, vs, side="right").astype(jnp.int32)
    deg = end - start
    E2 = 2 * N_EDGES

    def cond(state):
        remaining, partner, escore, cur, go = state
        return go

    def body(state):
        remaining, partner, escore, cur, go = state
        any_adv = jnp.bool_(False)
        for _ in range(8):
            valid = cur < deg
            posc = jnp.clip(start + cur, 0, E2 - 1)
            co = other_s[posc]
            dead = remaining & valid & ~remaining[co]
            cur = cur + dead.astype(jnp.int32)
            any_adv |= jnp.any(dead)
        valid = cur < deg
        posc = jnp.clip(start + cur, 0, E2 - 1)
        co = other_s[posc]
        ce = eid_s[posc]
        cand_alive = remaining & valid & remaining[co]
        matched = cand_alive & (ce[co] == ce)
        partner = jnp.where(matched, co, partner)
        escore = jnp.where(matched, e[ce], escore)
        remaining = remaining & ~matched
        return (remaining, partner, escore, cur,
                any_adv | jnp.any(matched))

    remaining0 = jnp.ones((N_NODES,), bool)
    partner0 = jnp.arange(N_NODES, dtype=jnp.int32)
    escore0 = jnp.ones((N_NODES,), e.dtype)
    cur0 = jnp.zeros((N_NODES,), jnp.int32)
    remaining, partner, escore, _, _ = jax.lax.while_loop(
        cond, body, (remaining0, partner0, escore0, cur0, jnp.bool_(True)))
    return partner, escore


def _clusters(partner, escore):
    v = jnp.arange(N_NODES, dtype=jnp.int32)
    rep = v <= partner
    cid = jnp.cumsum(rep.astype(jnp.int32)) - 1
    C = jnp.sum(rep.astype(jnp.int32))
    nodeA = jnp.zeros((N_NODES,), jnp.int32).at[
        jnp.where(rep, cid, N_NODES)].set(v, mode="drop")
    nodeB = partner[nodeA]
    sc = escore[nodeA]
    rowscale = jnp.where(jnp.arange(N_NODES) < C,
                         jnp.where(nodeB == nodeA, sc * 0.5, sc), 0.0)
    return nodeA, nodeB, rowscale, C


TBLK = 1024


def _pre_body(nx_ref, rs_ref, w2_ref, b2_ref, c_ref, pre_ref, cs_ref, acc):
    i = pl.program_id(0)
    nsteps = pl.num_programs(0)
    rs = rs_ref[...][:, 0:1]
    pre = _bdot(nx_ref[...] * rs, w2_ref[...]) + b2_ref[0:1, :]
    pre_ref[...] = pre

    @pl.when(i == 0)
    def _():
        acc[...] = jnp.zeros_like(acc)

    rid = i * TBLK + jax.lax.broadcasted_iota(jnp.int32, (TBLK, 1), 0)
    m = rid < c_ref[0, 0]
    acc[...] += jnp.sum(jnp.where(m, pre, 0.0), axis=0, keepdims=True)

    @pl.when(i == nsteps - 1)
    def _():
        cs_ref[...] = jnp.broadcast_to(acc[...], cs_ref.shape)


def _pre_pass(nx, rs128, W2, b2r, c11):
    return pl.pallas_call(
        _pre_body,
        grid=(CROWS // TBLK,),
        in_specs=[
            pl.BlockSpec((TBLK, D_H), lambda i: (i, 0)),
            pl.BlockSpec((TBLK, 128), lambda i: (i, 0)),
            pl.BlockSpec((D_H, 512), lambda i: (0, 0)),
            pl.BlockSpec((1, 512), lambda i: (0, 0)),
            pl.BlockSpec(memory_space=pltpu.SMEM),
        ],
        out_specs=[
            pl.BlockSpec((TBLK, 512), lambda i: (i, 0)),
            pl.BlockSpec((8, 512), lambda i: (0, 0)),
        ],
        out_shape=[
            jax.ShapeDtypeStruct((CROWS, 512), jnp.float32),
            jax.ShapeDtypeStruct((8, 512), jnp.float32),
        ],
        scratch_shapes=[pltpu.VMEM((1, 512), jnp.float32)],
    )(nx, rs128, W2, b2r, c11)


def _var2_body(pre_ref, mean_ref, c_ref, var_ref, acc):
    i = pl.program_id(0)
    nsteps = pl.num_programs(0)

    @pl.when(i == 0)
    def _():
        acc[...] = jnp.zeros_like(acc)

    rid = i * TBLK + jax.lax.broadcasted_iota(jnp.int32, (TBLK, 1), 0)
    m = rid < c_ref[0, 0]
    d = pre_ref[...] - mean_ref[0:1, :]
    acc[...] += jnp.sum(jnp.where(m, d * d, 0.0), axis=0, keepdims=True)

    @pl.when(i == nsteps - 1)
    def _():
        var_ref[...] = jnp.broadcast_to(
            acc[...] / c_ref[0, 0].astype(jnp.float32), var_ref.shape)


def _var2_pass(pre, mean, c11):
    return pl.pallas_call(
        _var2_body,
        grid=(CROWS // TBLK,),
        in_specs=[
            pl.BlockSpec((TBLK, 512), lambda i: (i, 0)),
            pl.BlockSpec((1, 512), lambda i: (0, 0)),
            pl.BlockSpec(memory_space=pltpu.SMEM),
        ],
        out_specs=pl.BlockSpec((8, 512), lambda i: (0, 0)),
        out_shape=jax.ShapeDtypeStruct((8, 512), jnp.float32),
        scratch_shapes=[pltpu.VMEM((1, 512), jnp.float32)],
    )(pre, mean, c11)


def _pool_body(pre_ref, mean_ref, var_ref, g2_ref, be2_ref, oh_ref,
               wfc_ref, bfc_ref, wfc1_ref, bfc1_ref, z_ref, pacc, cacc):
    i = pl.program_id(0)
    nsteps = pl.num_programs(0)

    @pl.when(i == 0)
    def _():
        pacc[...] = jnp.zeros_like(pacc)
        cacc[...] = jnp.zeros_like(cacc)

    h2 = jnp.maximum(
        g2_ref[0:1, :] * (pre_ref[...] - mean_ref[0:1, :])
        / jnp.sqrt(var_ref[0:1, :] + EPS) + be2_ref[0:1, :], 0.0)
    oh = oh_ref[...]
    pacc[...] += jax.lax.dot_general(
        oh, h2, (((0,), (0,)), ((), ())),
        preferred_element_type=jnp.float32,
        precision=jax.lax.Precision.HIGHEST)
    cacc[...] += jax.lax.dot_general(
        oh, jnp.ones((TBLK, 1), jnp.float32), (((0,), (0,)), ((), ())),
        preferred_element_type=jnp.float32,
        precision=jax.lax.Precision.HIGHEST)

    @pl.when(i == nsteps - 1)
    def _():
        pooled = pacc[...] / jnp.maximum(cacc[...], 1.0)
        z1 = jnp.maximum(_bdot(pooled, wfc_ref[...]) + bfc_ref[0:1, :], 0.0)
        z2 = _bdot(z1, wfc1_ref[...]) + bfc1_ref[0:1, :]
        z_ref[...] = z2[0:N_GRAPHS, :]


def _pool_pass(pre, mean, var, g2r, be2r, oh, Wfc, bfcr, Wfc1, bfc1r):
    return pl.pallas_call(
        _pool_body,
        grid=(CROWS // TBLK,),
        in_specs=[
            pl.BlockSpec((TBLK, 512), lambda i: (i, 0)),
            pl.BlockSpec((1, 512), lambda i: (0, 0)),
            pl.BlockSpec((1, 512), lambda i: (0, 0)),
            pl.BlockSpec((1, 512), lambda i: (0, 0)),
            pl.BlockSpec((1, 512), lambda i: (0, 0)),
            pl.BlockSpec((TBLK, 128), lambda i: (i, 0)),
            pl.BlockSpec((512, 200), lambda i: (0, 0)),
            pl.BlockSpec((1, 200), lambda i: (0, 0)),
            pl.BlockSpec((200, 10), lambda i: (0, 0)),
            pl.BlockSpec((1, 10), lambda i: (0, 0)),
        ],
        out_specs=pl.BlockSpec((N_GRAPHS, 10), lambda i: (0, 0)),
        out_shape=jax.ShapeDtypeStruct((N_GRAPHS, 10), jnp.float32),
        scratch_shapes=[
            pltpu.VMEM((128, 512), jnp.float32),
            pltpu.VMEM((128, 1), jnp.float32),
        ],
    )(pre, mean, var, g2r, be2r, oh, Wfc, bfcr, Wfc1, bfc1r)


def kernel(x, edge_index, batch, W1, b1, g1, be1, Wp, bp, W2, b2, g2, be2,
           Wfc, bfc, Wfc1, bfc1):
    src = edge_index[0]
    dst = edge_index[1]

    b1r = b1.reshape(1, D_H)
    g1r = g1.reshape(1, D_H)
    be1r = be1.reshape(1, D_H)
    y, cs8 = _y_pass(x, W1, b1r)
    mean = (cs8[0:1] / jnp.float32(N_NODES))
    var8 = _var_pass(y, mean)

    Wp2 = jnp.zeros((D_H, 128), jnp.float32)
    Wp2 = Wp2.at[:, 0].set(Wp[:D_H, 0]).at[:, 1].set(Wp[D_H:, 0])
    h, ppad = _h_and_p(y, mean, var8[0:1], g1r, be1r, Wp2)
    p0 = ppad[:, 0]
    p1 = ppad[:, 1]

    padi = jnp.full((E_PAD - N_EDGES,), N_NODES, jnp.int32)
    srcm = jnp.concatenate([src, padi]).reshape(SC_TILES * EROWS, 128)
    dstm = jnp.concatenate([dst, padi]).reshape(SC_TILES * EROWS, 128)
    padp = jnp.full((NP - N_NODES,), -1000.0, jnp.float32)
    p0p = jnp.concatenate([p0 + bp[0], padp])
    p1p = jnp.concatenate([p1, padp])
    e = _sc_edge(srcm, dstm, p0p, p1p).reshape(E_PAD)[:N_EDGES]

    partner, escore = _merge_rounds(e, src, dst)
    nodeA, nodeB, rowscale, C = _clusters(partner, escore)

    padc = jnp.zeros((CROWS - N_NODES,), jnp.int32)
    iap = jnp.concatenate([nodeA, padc])
    ibp = jnp.concatenate([nodeB, padc])
    new_x_raw = _sc_newx(h, iap, ibp)

    rsp = jnp.concatenate([rowscale, padc.astype(jnp.float32)])
    rs128 = jnp.broadcast_to(rsp[:, None], (CROWS, 128))
    valid = jnp.arange(N_NODES) < C
    new_batch = jnp.where(valid, jnp.maximum(batch[nodeA], batch[nodeB]),
                          N_GRAPHS)
    nbp = jnp.concatenate([new_batch,
                           jnp.full((CROWS - N_NODES,), N_GRAPHS, jnp.int32)])
    oh = jnp.zeros((CROWS, 128), jnp.float32).at[:, :N_GRAPHS].set(
        (nbp[:, None] == jnp.arange(N_GRAPHS)[None, :]).astype(jnp.float32))

    c11 = C.reshape(1, 1)
    pre, cs8 = _pre_pass(new_x_raw, rs128, W2, b2.reshape(1, 512), c11)
    mean2 = cs8[0:1] / C.astype(jnp.float32)
    var2 = _var2_pass(pre, mean2, c11)
    return _pool_pass(pre, mean2, var2[0:1], g2.reshape(1, 512),
                      be2.reshape(1, 512), oh, Wfc, bfc.reshape(1, 200),
                      Wfc1, bfc1.reshape(1, 10))
